# baseline (device time: 52338 ns/iter reference)
import jax
import jax.numpy as jnp
from jax import lax
from jax.experimental import pallas as pl
from jax.experimental.pallas import tpu as pltpu

N_DEV = 4
B, SQ, SKV, HQ, DH = 2, 128, 512, 16, 64
H_LOC = HQ // N_DEV
SKV_LOC = SKV // N_DEV
DM = 512
BF16 = jnp.bfloat16
F32 = jnp.float32


def kernel(x, Wq, K_ext, V_ext, Wo):
    def body(x_ref, wq_ref, k_ref, v_ref, wo_ref, out_ref,
             k_all, v_all, o_buf,
             k_send_sems, v_send_sems, k_recv_sems, v_recv_sems,
             o_send_sems, o_recv_sems, local_sems):
        my = lax.axis_index("i")

        barrier = pltpu.get_barrier_semaphore()
        for off in range(1, N_DEV):
            peer = lax.rem(my + off, N_DEV)
            pl.semaphore_signal(barrier, inc=1, device_id=(peer,),
                                device_id_type=pl.DeviceIdType.MESH)
        pl.semaphore_wait(barrier, N_DEV - 1)

        kv_sends = []
        for off in range(1, N_DEV):
            o = off - 1
            peer = lax.rem(my + off, N_DEV)
            for ref, all_ref, ssems, rsems in (
                (k_ref, k_all, k_send_sems, k_recv_sems),
                (v_ref, v_all, v_send_sems, v_recv_sems),
            ):
                rdma = pltpu.make_async_remote_copy(
                    src_ref=ref.at[:, :, pl.ds(peer * H_LOC, H_LOC), :],
                    dst_ref=all_ref.at[:, pl.ds(my * SKV_LOC, SKV_LOC), :, :],
                    send_sem=ssems.at[o],
                    recv_sem=rsems.at[o],
                    device_id=(peer,),
                    device_id_type=pl.DeviceIdType.MESH,
                )
                rdma.start()
                kv_sends.append(rdma)

        cp_k = pltpu.make_async_copy(
            k_ref.at[:, :, pl.ds(my * H_LOC, H_LOC), :],
            k_all.at[:, pl.ds(my * SKV_LOC, SKV_LOC), :, :],
            local_sems.at[0],
        )
        cp_v = pltpu.make_async_copy(
            v_ref.at[:, :, pl.ds(my * H_LOC, H_LOC), :],
            v_all.at[:, pl.ds(my * SKV_LOC, SKV_LOC), :, :],
            local_sems.at[1],
        )
        cp_k.start()
        cp_v.start()

        x2d = x_ref[...].reshape(B * SQ, DM).astype(BF16)
        wq = wq_ref[...].astype(BF16)
        q2d = jax.lax.dot_general(
            x2d, wq, (((1,), (0,)), ((), ())), preferred_element_type=F32
        )

        cp_k.wait()
        cp_v.wait()

        for off in range(1, N_DEV):
            o = off - 1
            src = lax.rem(my + N_DEV - off, N_DEV)
            for all_ref, ssems, rsems in (
                (k_all, k_send_sems, k_recv_sems),
                (v_all, v_send_sems, v_recv_sems),
            ):
                dst = all_ref.at[:, pl.ds(src * SKV_LOC, SKV_LOC), :, :]
                recv = pltpu.make_async_remote_copy(
                    src_ref=dst, dst_ref=dst,
                    send_sem=ssems.at[o],
                    recv_sem=rsems.at[o],
                    device_id=(src,),
                    device_id_type=pl.DeviceIdType.MESH,
                )
                recv.wait_recv()

        row = lax.broadcasted_iota(jnp.int32, (SQ, SKV), 0)
        col = lax.broadcasted_iota(jnp.int32, (SQ, SKV), 1)
        qb = row // 64
        kb = col // 64
        mask = (qb == kb) | ((kb % 4) == (qb % 4))

        ctx_rows = []
        for b in range(B):
            ctx_h = []
            for h in range(H_LOC):
                q_bh = q2d[b * SQ:(b + 1) * SQ, h * DH:(h + 1) * DH].astype(BF16)
                k_bh = k_all[b, :, h, :].astype(BF16)
                v_bh = v_all[b, :, h, :].astype(BF16)
                scores = lax.dot_general(
                    q_bh, k_bh, (((1,), (1,)), ((), ())),
                    preferred_element_type=F32,
                ) * 0.125
                scores = jnp.where(mask, scores, -1e9)
                m = jnp.max(scores, axis=-1, keepdims=True)
                w = jnp.exp(scores - m)
                w = w / jnp.sum(w, axis=-1, keepdims=True)
                ctx_bh = lax.dot_general(
                    w.astype(BF16), v_bh, (((1,), (0,)), ((), ())),
                    preferred_element_type=F32,
                )
                ctx_h.append(ctx_bh)
            ctx_rows.append(jnp.concatenate(ctx_h, axis=1))
        ctx2d = jnp.concatenate(ctx_rows, axis=0)

        wo = wo_ref[...].astype(BF16)
        partial = lax.dot_general(
            ctx2d.astype(BF16), wo, (((1,), (0,)), ((), ())),
            preferred_element_type=F32,
        )
        partial = partial.reshape(B, SQ, DM)
        out_ref[...] = partial

        o_sends = []
        for off in range(1, N_DEV):
            o = off - 1
            peer = lax.rem(my + off, N_DEV)
            rdma = pltpu.make_async_remote_copy(
                src_ref=out_ref,
                dst_ref=o_buf.at[o],
                send_sem=o_send_sems.at[o],
                recv_sem=o_recv_sems.at[o],
                device_id=(peer,),
                device_id_type=pl.DeviceIdType.MESH,
            )
            rdma.start()
            o_sends.append(rdma)

        for rdma in kv_sends:
            rdma.wait_send()

        acc = partial
        for o in range(N_DEV - 1):
            recv = pltpu.make_async_remote_copy(
                src_ref=o_buf.at[o], dst_ref=o_buf.at[o],
                send_sem=o_send_sems.at[o],
                recv_sem=o_recv_sems.at[o],
                device_id=(my,),
                device_id_type=pl.DeviceIdType.MESH,
            )
            recv.wait_recv()
            acc = acc + o_buf[o]

        for rdma in o_sends:
            rdma.wait_send()
        out_ref[...] = acc

    return pl.pallas_call(
        body,
        out_shape=jax.ShapeDtypeStruct((B, SQ, DM), F32),
        in_specs=[pl.BlockSpec(memory_space=pltpu.VMEM)] * 5,
        out_specs=pl.BlockSpec(memory_space=pltpu.VMEM),
        scratch_shapes=[
            pltpu.VMEM((B, SKV, H_LOC, DH), F32),
            pltpu.VMEM((B, SKV, H_LOC, DH), F32),
            pltpu.VMEM((N_DEV - 1, B, SQ, DM), F32),
            pltpu.SemaphoreType.DMA((N_DEV - 1,)),
            pltpu.SemaphoreType.DMA((N_DEV - 1,)),
            pltpu.SemaphoreType.DMA((N_DEV - 1,)),
            pltpu.SemaphoreType.DMA((N_DEV - 1,)),
            pltpu.SemaphoreType.DMA((N_DEV - 1,)),
            pltpu.SemaphoreType.DMA((N_DEV - 1,)),
            pltpu.SemaphoreType.DMA((2,)),
        ],
        compiler_params=pltpu.CompilerParams(collective_id=0),
    )(x, Wq, K_ext, V_ext, Wo)


# device time: 30671 ns/iter; 1.7064x vs baseline; 1.7064x over previous
import jax
import jax.numpy as jnp
from jax import lax
from jax.experimental import pallas as pl
from jax.experimental.pallas import tpu as pltpu

N_DEV = 4
B, SQ, SKV, HQ, DH = 2, 128, 512, 16, 64
H_LOC = HQ // N_DEV
SKV_LOC = SKV // N_DEV
SKV_USE = 2 * SKV_LOC
DM = 512
BF16 = jnp.bfloat16
F32 = jnp.float32


def kernel(x, Wq, K_ext, V_ext, Wo):
    def body(x_ref, wq_ref, k_ref, v_ref, wo_ref, out_ref,
             k_need, v_need, k_stage, v_stage, o_stage, o_buf,
             k_send_sems, v_send_sems, k_recv_sems, v_recv_sems,
             o_send_sems, o_recv_sems):
        my = lax.axis_index("i")
        i_am_kv_owner = lax.rem(my, 2) == 0
        half = lax.div(my, 2)

        barrier = pltpu.get_barrier_semaphore()
        for off in range(1, N_DEV):
            peer = lax.rem(my + off, N_DEV)
            pl.semaphore_signal(barrier, inc=1, device_id=(peer,),
                                device_id_type=pl.DeviceIdType.MESH)
        pl.semaphore_wait(barrier, N_DEV - 1)

        kv_sends = []
        for off in range(1, N_DEV):
            o = off - 1
            peer = lax.rem(my + off, N_DEV)
            for stage, need, ssems, rsems in (
                (k_stage, k_need, k_send_sems, k_recv_sems),
                (v_stage, v_need, v_send_sems, v_recv_sems),
            ):
                rdma = pltpu.make_async_remote_copy(
                    src_ref=stage.at[o],
                    dst_ref=need.at[:, pl.ds(half * SKV_LOC, SKV_LOC), :, :],
                    send_sem=ssems.at[o],
                    recv_sem=rsems.at[half],
                    device_id=(peer,),
                    device_id_type=pl.DeviceIdType.MESH,
                )
                kv_sends.append(rdma)

        @pl.when(i_am_kv_owner)
        def _send_kv():
            for off in range(1, N_DEV):
                o = off - 1
                peer = lax.rem(my + off, N_DEV)
                k_stage[o] = k_ref[:, :, pl.ds(peer * H_LOC, H_LOC), :].astype(BF16)
                v_stage[o] = v_ref[:, :, pl.ds(peer * H_LOC, H_LOC), :].astype(BF16)
            for rdma in kv_sends:
                rdma.start()
            k_need[:, pl.ds(half * SKV_LOC, SKV_LOC), :, :] = (
                k_ref[:, :, pl.ds(my * H_LOC, H_LOC), :].astype(BF16))
            v_need[:, pl.ds(half * SKV_LOC, SKV_LOC), :, :] = (
                v_ref[:, :, pl.ds(my * H_LOC, H_LOC), :].astype(BF16))

        x2d = x_ref[...].reshape(B * SQ, DM).astype(BF16)
        wq = wq_ref[...].astype(BF16)
        q2d = lax.dot_general(
            x2d, wq, (((1,), (0,)), ((), ())), preferred_element_type=F32
        )

        for h in range(2):
            @pl.when(my != 2 * h)
            def _wait_kv(h=h):
                for need, ssems, rsems in (
                    (k_need, k_send_sems, k_recv_sems),
                    (v_need, v_send_sems, v_recv_sems),
                ):
                    dst = need.at[:, pl.ds(h * SKV_LOC, SKV_LOC), :, :]
                    recv = pltpu.make_async_remote_copy(
                        src_ref=dst, dst_ref=dst,
                        send_sem=ssems.at[0],
                        recv_sem=rsems.at[h],
                        device_id=(2 * h,),
                        device_id_type=pl.DeviceIdType.MESH,
                    )
                    recv.wait_recv()

        row = lax.broadcasted_iota(jnp.int32, (SQ, SKV_USE), 0)
        col = lax.broadcasted_iota(jnp.int32, (SQ, SKV_USE), 1)
        mask = ((col // 64) % 2) == (row // 64)

        ctx_rows = []
        for b in range(B):
            ctx_h = []
            for h in range(H_LOC):
                q_bh = q2d[b * SQ:(b + 1) * SQ, h * DH:(h + 1) * DH].astype(BF16)
                k_bh = k_need[b, :, h, :]
                v_bh = v_need[b, :, h, :]
                scores = lax.dot_general(
                    q_bh, k_bh, (((1,), (1,)), ((), ())),
                    preferred_element_type=F32,
                ) * 0.125
                scores = jnp.where(mask, scores, -1e9)
                m = jnp.max(scores, axis=-1, keepdims=True)
                w = jnp.exp(scores - m)
                w = w / jnp.sum(w, axis=-1, keepdims=True)
                ctx_bh = lax.dot_general(
                    w.astype(BF16), v_bh, (((1,), (0,)), ((), ())),
                    preferred_element_type=F32,
                )
                ctx_h.append(ctx_bh)
            ctx_rows.append(jnp.concatenate(ctx_h, axis=1))
        ctx2d = jnp.concatenate(ctx_rows, axis=0)

        wo = wo_ref[...].astype(BF16)
        partial = lax.dot_general(
            ctx2d.astype(BF16), wo, (((1,), (0,)), ((), ())),
            preferred_element_type=F32,
        )
        o_stage[...] = partial.astype(BF16).reshape(B, SQ, DM)

        o_sends = []
        for off in range(1, N_DEV):
            o = off - 1
            peer = lax.rem(my + off, N_DEV)
            rdma = pltpu.make_async_remote_copy(
                src_ref=o_stage,
                dst_ref=o_buf.at[o],
                send_sem=o_send_sems.at[o],
                recv_sem=o_recv_sems.at[o],
                device_id=(peer,),
                device_id_type=pl.DeviceIdType.MESH,
            )
            rdma.start()
            o_sends.append(rdma)

        @pl.when(i_am_kv_owner)
        def _retire_kv_sends():
            for rdma in kv_sends:
                rdma.wait_send()

        acc = partial.reshape(B, SQ, DM)
        for o in range(N_DEV - 1):
            recv = pltpu.make_async_remote_copy(
                src_ref=o_buf.at[o], dst_ref=o_buf.at[o],
                send_sem=o_send_sems.at[o],
                recv_sem=o_recv_sems.at[o],
                device_id=(my,),
                device_id_type=pl.DeviceIdType.MESH,
            )
            recv.wait_recv()
            acc = acc + o_buf[o].astype(F32)
        out_ref[...] = acc

        for rdma in o_sends:
            rdma.wait_send()

    return pl.pallas_call(
        body,
        out_shape=jax.ShapeDtypeStruct((B, SQ, DM), F32),
        in_specs=[pl.BlockSpec(memory_space=pltpu.VMEM)] * 5,
        out_specs=pl.BlockSpec(memory_space=pltpu.VMEM),
        scratch_shapes=[
            pltpu.VMEM((B, SKV_USE, H_LOC, DH), BF16),
            pltpu.VMEM((B, SKV_USE, H_LOC, DH), BF16),
            pltpu.VMEM((N_DEV - 1, B, SKV_LOC, H_LOC, DH), BF16),
            pltpu.VMEM((N_DEV - 1, B, SKV_LOC, H_LOC, DH), BF16),
            pltpu.VMEM((B, SQ, DM), BF16),
            pltpu.VMEM((N_DEV - 1, B, SQ, DM), BF16),
            pltpu.SemaphoreType.DMA((N_DEV - 1,)),
            pltpu.SemaphoreType.DMA((N_DEV - 1,)),
            pltpu.SemaphoreType.DMA((2,)),
            pltpu.SemaphoreType.DMA((2,)),
            pltpu.SemaphoreType.DMA((N_DEV - 1,)),
            pltpu.SemaphoreType.DMA((N_DEV - 1,)),
        ],
        compiler_params=pltpu.CompilerParams(collective_id=0),
    )(x, Wq, K_ext, V_ext, Wo)


# device time: 20337 ns/iter; 2.5735x vs baseline; 1.5081x over previous
import jax
import jax.numpy as jnp
from jax import lax
from jax.experimental import pallas as pl
from jax.experimental.pallas import tpu as pltpu

N_DEV = 4
B, SQ, SKV, HQ, DH = 2, 128, 512, 16, 64
H_LOC = HQ // N_DEV
SKV_LOC = SKV // N_DEV
SKV_USE = 2 * SKV_LOC
DM = 512
BF16 = jnp.bfloat16
F32 = jnp.float32


def kernel(x, Wq, K_ext, V_ext, Wo):
    K_ext = K_ext.reshape(B, SKV_LOC, HQ * DH)
    V_ext = V_ext.reshape(B, SKV_LOC, HQ * DH)

    def body(x_ref, wq_ref, k_ref, v_ref, wo_ref, out_ref,
             k_need, v_need, k_stage, v_stage, o_stage, o_buf,
             k_send_sems, v_send_sems, k_recv_sems, v_recv_sems,
             o_send_sems, o_recv_sems):
        my = lax.axis_index("i")
        i_am_kv_owner = lax.rem(my, 2) == 0
        half = lax.div(my, 2)

        barrier = pltpu.get_barrier_semaphore()
        for off in range(1, N_DEV):
            peer = lax.rem(my + off, N_DEV)
            pl.semaphore_signal(barrier, inc=1, device_id=(peer,),
                                device_id_type=pl.DeviceIdType.MESH)

        @pl.when(i_am_kv_owner)
        def _stage_kv():
            for off in range(1, N_DEV):
                o = off - 1
                peer = lax.rem(my + off, N_DEV)
                k_stage[o] = k_ref[:, :, pl.ds(peer * H_LOC * DH, H_LOC * DH)].astype(BF16)
                v_stage[o] = v_ref[:, :, pl.ds(peer * H_LOC * DH, H_LOC * DH)].astype(BF16)
            k_need[:, pl.ds(half * SKV_LOC, SKV_LOC), :] = (
                k_ref[:, :, pl.ds(my * H_LOC * DH, H_LOC * DH)].astype(BF16))
            v_need[:, pl.ds(half * SKV_LOC, SKV_LOC), :] = (
                v_ref[:, :, pl.ds(my * H_LOC * DH, H_LOC * DH)].astype(BF16))

        pl.semaphore_wait(barrier, N_DEV - 1)

        SEND_ORDER = (1, 2, 3)
        kv_sends = []
        for stage, need, ssems, rsems in (
            (k_stage, k_need, k_send_sems, k_recv_sems),
            (v_stage, v_need, v_send_sems, v_recv_sems),
        ):
            for off in SEND_ORDER:
                o = off - 1
                peer = lax.rem(my + off, N_DEV)
                kv_sends.append(pltpu.make_async_remote_copy(
                    src_ref=stage.at[o],
                    dst_ref=need.at[:, pl.ds(half * SKV_LOC, SKV_LOC), :],
                    send_sem=ssems.at[o],
                    recv_sem=rsems.at[half],
                    device_id=(peer,),
                    device_id_type=pl.DeviceIdType.MESH,
                ))
        k_sends, v_sends = kv_sends[:3], kv_sends[3:]

        @pl.when(i_am_kv_owner)
        def _send_kv():
            for rdma in k_sends:
                rdma.start()
            for rdma in v_sends:
                rdma.start()

        x2d = x_ref[...].reshape(B * SQ, DM).astype(BF16)
        wq = wq_ref[...].astype(BF16)
        q2d = lax.dot_general(
            x2d, wq, (((1,), (0,)), ((), ())), preferred_element_type=F32
        )
        wo = wo_ref[...].astype(BF16)
        row = lax.broadcasted_iota(jnp.int32, (SQ, SKV_USE), 0)
        col = lax.broadcasted_iota(jnp.int32, (SQ, SKV_USE), 1)
        mask = ((col // 64) % 2) == (row // 64)

        def wait_half(need, ssems, rsems, h):
            dst = need.at[:, pl.ds(h * SKV_LOC, SKV_LOC), :]
            recv = pltpu.make_async_remote_copy(
                src_ref=dst, dst_ref=dst,
                send_sem=ssems.at[0],
                recv_sem=rsems.at[h],
                device_id=(2 * h,),
                device_id_type=pl.DeviceIdType.MESH,
            )
            recv.wait_recv()

        for h in range(2):
            @pl.when(my != 2 * h)
            def _wait_k(h=h):
                wait_half(k_need, k_send_sems, k_recv_sems, h)

        w_all = []
        for b in range(B):
            for h in range(H_LOC):
                q_bh = q2d[b * SQ:(b + 1) * SQ, h * DH:(h + 1) * DH].astype(BF16)
                k_bh = k_need[b, :, h * DH:(h + 1) * DH]
                scores = lax.dot_general(
                    q_bh, k_bh, (((1,), (1,)), ((), ())),
                    preferred_element_type=F32,
                ) * 0.125
                w = jnp.where(mask, jnp.exp(scores), 0.0)
                w_all.append((w / jnp.sum(w, axis=-1, keepdims=True)).astype(BF16))

        for h in range(2):
            @pl.when(my != 2 * h)
            def _wait_v(h=h):
                wait_half(v_need, v_send_sems, v_recv_sems, h)

        o_sends = []
        partials = []
        for b in range(B):
            ctx_h = []
            for h in range(H_LOC):
                v_bh = v_need[b, :, h * DH:(h + 1) * DH]
                ctx_h.append(lax.dot_general(
                    w_all[b * H_LOC + h], v_bh, (((1,), (0,)), ((), ())),
                    preferred_element_type=F32,
                ))
            ctx_b = jnp.concatenate(ctx_h, axis=1)
            partial_b = lax.dot_general(
                ctx_b.astype(BF16), wo, (((1,), (0,)), ((), ())),
                preferred_element_type=F32,
            )
            partials.append(partial_b)
            o_stage[b] = partial_b.astype(BF16)
            for off in SEND_ORDER:
                o = off - 1
                peer = lax.rem(my + off, N_DEV)
                rdma = pltpu.make_async_remote_copy(
                    src_ref=o_stage.at[b],
                    dst_ref=o_buf.at[o, b],
                    send_sem=o_send_sems.at[o, b],
                    recv_sem=o_recv_sems.at[o, b],
                    device_id=(peer,),
                    device_id_type=pl.DeviceIdType.MESH,
                )
                rdma.start()
                o_sends.append(rdma)

        @pl.when(i_am_kv_owner)
        def _retire_kv_sends():
            for rdma in kv_sends:
                rdma.wait_send()

        for b in range(B):
            acc = partials[b]
            for o in range(N_DEV - 1):
                recv = pltpu.make_async_remote_copy(
                    src_ref=o_buf.at[o, b], dst_ref=o_buf.at[o, b],
                    send_sem=o_send_sems.at[o, b],
                    recv_sem=o_recv_sems.at[o, b],
                    device_id=(my,),
                    device_id_type=pl.DeviceIdType.MESH,
                )
                recv.wait_recv()
                acc = acc + o_buf[o, b].astype(F32)
            out_ref[b] = acc

        for rdma in o_sends:
            rdma.wait_send()

    return pl.pallas_call(
        body,
        out_shape=jax.ShapeDtypeStruct((B, SQ, DM), F32),
        in_specs=[pl.BlockSpec(memory_space=pltpu.VMEM)] * 5,
        out_specs=pl.BlockSpec(memory_space=pltpu.VMEM),
        scratch_shapes=[
            pltpu.VMEM((B, SKV_USE, H_LOC * DH), BF16),
            pltpu.VMEM((B, SKV_USE, H_LOC * DH), BF16),
            pltpu.VMEM((N_DEV - 1, B, SKV_LOC, H_LOC * DH), BF16),
            pltpu.VMEM((N_DEV - 1, B, SKV_LOC, H_LOC * DH), BF16),
            pltpu.VMEM((B, SQ, DM), BF16),
            pltpu.VMEM((N_DEV - 1, B, SQ, DM), BF16),
            pltpu.SemaphoreType.DMA((N_DEV - 1,)),
            pltpu.SemaphoreType.DMA((N_DEV - 1,)),
            pltpu.SemaphoreType.DMA((2,)),
            pltpu.SemaphoreType.DMA((2,)),
            pltpu.SemaphoreType.DMA((N_DEV - 1, B)),
            pltpu.SemaphoreType.DMA((N_DEV - 1, B)),
        ],
        compiler_params=pltpu.CompilerParams(collective_id=0),
    )(x, Wq, K_ext, V_ext, Wo)
